# baseline (speedup 1.0000x reference)
import functools

import jax
import jax.numpy as jnp
from jax.experimental import pallas as pl
from jax.experimental.pallas import tpu as pltpu

_GROUP = 4


def _mlp_body(n_slabs, x_ref, valid_ref, w1_ref, b1_ref, w2_ref, b2_ref, out_ref):
    x = jnp.maximum(x_ref[...], 0.0)
    acc = None
    for g in range(n_slabs):
        xg = x[:, 128 * g:128 * (g + 1)]
        hg = jnp.dot(xg, w1_ref[g], preferred_element_type=jnp.float32)
        hg = jnp.maximum(hg + b1_ref[g][None, :], 0.0)
        cg = jnp.dot(hg, w2_ref[g], preferred_element_type=jnp.float32)
        acc = cg if acc is None else acc + cg
    out = acc + b2_ref[...]
    out_ref[...] = jnp.where(valid_ref[...] > 0, out, 0.0)


@functools.partial(jax.jit, static_argnames=("block_rows",))
def _run(h, valid, W1, b1, W2, b2, block_rows=2048):
    B, N, C = h.shape
    Wh = W1.shape[2]
    G = _GROUP
    S = N // G
    lanes = G * C

    x = h.reshape(B, N * C)

    eye_g = jnp.eye(G, dtype=W1.dtype)
    W1s = W1.reshape(S, G, C, Wh)
    W1bd = (W1s[:, :, :, None, :] * eye_g[None, :, None, :, None]).reshape(S, G * C, G * Wh)
    b1p = b1.reshape(S, G * Wh)

    W2s = W2[:, :, 0].reshape(S, G, Wh)
    eye_n = jnp.eye(N, dtype=W2.dtype).reshape(S, G, 1, N)
    W2p = (W2s[:, :, :, None] * eye_n).reshape(S, G * Wh, N)
    b2p = b2[:, 0][None, :]

    grid = (B // block_rows,)
    out = pl.pallas_call(
        functools.partial(_mlp_body, S),
        grid=grid,
        in_specs=[
            pl.BlockSpec((block_rows, N * C), lambda i: (i, 0)),
            pl.BlockSpec((block_rows, N), lambda i: (i, 0)),
            pl.BlockSpec((S, lanes, lanes), lambda i: (0, 0, 0)),
            pl.BlockSpec((S, lanes), lambda i: (0, 0)),
            pl.BlockSpec((S, lanes, N), lambda i: (0, 0, 0)),
            pl.BlockSpec((1, N), lambda i: (0, 0)),
        ],
        out_specs=pl.BlockSpec((block_rows, N), lambda i: (i, 0)),
        out_shape=jax.ShapeDtypeStruct((B, N), jnp.float32),
        compiler_params=pltpu.CompilerParams(
            dimension_semantics=("arbitrary",),
        ),
    )(x, valid, W1bd, b1p, W2p, b2p)
    return out.reshape(B, N, 1)


def kernel(h, valid, W1, b1, W2, b2):
    return _run(h, valid, W1, b1, W2, b2)
